# baseline (speedup 1.0000x reference)
import functools

import jax
import jax.numpy as jnp
from jax import lax
from jax.experimental import pallas as pl
from jax.experimental.pallas import tpu as pltpu
from jax.experimental.pallas import tpu_sc as plsc

N = 10000
E = 320000
D = 128

NC = 2
NS = 16
NW = NC * NS
EPW = E // NW
CH = 80
NCHUNK = EPW // CH
SCH = 125
SNCH = EPW // SCH
SIBL = 20
SNBLK = SNCH // SIBL
NP = 10240
DCH = 2560
ZR = CH
_NZB = -(-(N // ZR) // NS)

_mesh = plsc.VectorSubcoreMesh(
    core_axis_name="c", subcore_axis_name="s", num_cores=NC, num_subcores=NS)


def _deg_body(dst_hbm, degp_hbm, dstv2, onesv, zv, acc, sd):
    c = lax.axis_index("c")
    s = lax.axis_index("s")
    wid = c * NS + s
    zz = jnp.zeros((16,), jnp.float32)
    oo = jnp.ones((16,), jnp.float32)

    def fill(i, _):
        zv[pl.ds(i * 16, 16)] = zz
        return 0
    lax.fori_loop(0, DCH // 16, fill, 0)

    def fillo(i, _):
        onesv[pl.ds(i * 16, 16)] = oo
        return 0
    lax.fori_loop(0, CH // 16, fillo, 0)

    pltpu.sync_copy(dst_hbm.at[wid], dstv2)

    @pl.when(s < NP // DCH)
    def _():
        pltpu.sync_copy(zv, acc.at[pl.ds(s * DCH, DCH)])
    plsc.subcore_barrier()

    GRP = 25
    for g in range(NCHUNK // GRP):
        cps = [pltpu.async_copy(onesv, acc.at[dstv2.at[g * GRP + j]], sd,
                                add=True)
               for j in range(GRP)]
        for cp in cps:
            cp.wait()
    plsc.subcore_barrier()

    @pl.when(s < NP // DCH)
    def _():
        pltpu.sync_copy(acc.at[pl.ds(s * DCH, DCH)], zv)
        pltpu.sync_copy(zv, degp_hbm.at[pl.ds(c * NP + s * DCH, DCH)])


_deg_kernel = functools.partial(
    pl.kernel,
    out_type=jax.ShapeDtypeStruct((NC * NP,), jnp.float32),
    mesh=_mesh,
    scratch_types=[
        pltpu.VMEM((NCHUNK, CH), jnp.int32),
        pltpu.VMEM((CH,), jnp.float32),
        pltpu.VMEM((DCH,), jnp.float32),
        pltpu.VMEM_SHARED((NP,), jnp.float32),
        pltpu.SemaphoreType.DMA,
    ],
)(_deg_body)


_NB = 2


def _scat_body(g_hbm, src_hbm, dst_hbm, parts_hbm, srcv2, dstv2, rows,
               acc, sg, ss):
    c = lax.axis_index("c")
    s = lax.axis_index("s")
    wid = c * NS + s
    zz = jnp.zeros((16,), jnp.float32)

    def zero(i, _):
        rows[0, i // 8, pl.ds((i % 8) * 16, 16)] = zz
        return 0
    lax.fori_loop(0, ZR * 8, zero, 0)
    for j in range(_NZB):
        blk = s + NS * j

        @pl.when(blk < N // ZR)
        def _():
            pltpu.sync_copy(rows.at[0, pl.ds(0, ZR)],
                            acc.at[pl.ds(blk * ZR, ZR)])
    plsc.subcore_barrier()

    def gather(lk, b):
        return pltpu.async_copy(g_hbm.at[srcv2.at[lk]], rows.at[b], sg[b])

    def scat(lk, b):
        return pltpu.async_copy(rows.at[b], acc.at[dstv2.at[lk]], ss[b],
                                add=True)

    def wait_g(b):
        pltpu.make_async_copy(g_hbm.at[srcv2.at[0]], rows.at[b], sg[b]).wait()

    def wait_s(b):
        pltpu.make_async_copy(rows.at[b], acc.at[dstv2.at[0]], ss[b]).wait()

    for blk in range(SNBLK):
        pltpu.sync_copy(src_hbm.at[wid, blk], srcv2)
        pltpu.sync_copy(dst_hbm.at[wid, blk], dstv2)
        for b in range(_NB):
            gather(b, b)

        def body(k2, _):
            lk = k2 * _NB
            for b in range(_NB):
                wait_g(b)
                scat(lk + b, b)
                wait_s(b)
                nk = lk + _NB + b

                @pl.when(nk < SIBL)
                def _():
                    gather(nk, b)
            return 0
        lax.fori_loop(0, SIBL // _NB, body, 0)
        for t in range(SIBL - (SIBL // _NB) * _NB):
            wait_g(t)
            scat((SIBL // _NB) * _NB + t, t)
            wait_s(t)
    plsc.subcore_barrier()
    for j in range(_NZB):
        blk = s + NS * j

        @pl.when(blk < N // ZR)
        def _():
            pltpu.sync_copy(acc.at[pl.ds(blk * ZR, ZR)],
                            parts_hbm.at[c, pl.ds(blk * ZR, ZR)])


_scat_kernel = functools.partial(
    pl.kernel,
    out_type=jax.ShapeDtypeStruct((NC, N, D), jnp.float32),
    mesh=_mesh,
    scratch_types=[
        pltpu.VMEM((SIBL, SCH), jnp.int32),
        pltpu.VMEM((SIBL, SCH), jnp.int32),
        pltpu.VMEM((_NB, SCH, D), jnp.float32),
        pltpu.VMEM_SHARED((N, D), jnp.float32),
        [pltpu.SemaphoreType.DMA] * _NB,
        [pltpu.SemaphoreType.DMA] * _NB,
    ],
)(_scat_body)


def _dec_body(z_hbm, src_hbm, dst_hbm, out_hbm, srcv2, dstv2, za, zb, svall,
              sa, sb):
    c = lax.axis_index("c")
    s = lax.axis_index("s")
    wid = c * NS + s
    lane = lax.iota(jnp.int32, 16)

    pltpu.sync_copy(src_hbm.at[wid], srcv2)
    pltpu.sync_copy(dst_hbm.at[wid], dstv2)

    def ga(k, b):
        return pltpu.async_copy(z_hbm.at[srcv2.at[k]], za.at[b], sa[b])

    def gb(k, b):
        return pltpu.async_copy(z_hbm.at[dstv2.at[k]], zb.at[b], sb[b])

    def wait_ga(b):
        pltpu.make_async_copy(z_hbm.at[srcv2.at[0]], za.at[b], sa[b]).wait()

    def wait_gb(b):
        pltpu.make_async_copy(z_hbm.at[dstv2.at[0]], zb.at[b], sb[b]).wait()

    rots = [(lane + t) & 15 for t in (8, 4, 2, 1)]

    def compute(k, b):
        def grp(gi, _):
            def row(r16, vec):
                r = gi * 16 + r16
                p = za[b, r, pl.ds(0, 16)] * zb[b, r, pl.ds(0, 16)]
                q = za[b, r, pl.ds(64, 16)] * zb[b, r, pl.ds(64, 16)]
                for j in range(1, 4):
                    p = p + (za[b, r, pl.ds(j * 16, 16)]
                             * zb[b, r, pl.ds(j * 16, 16)])
                    q = q + (za[b, r, pl.ds(64 + j * 16, 16)]
                             * zb[b, r, pl.ds(64 + j * 16, 16)])
                p = p + q
                for rot in rots:
                    p = p + p.at[rot].get(mode="promise_in_bounds")
                return jnp.where(lane == r16, p, vec)
            vec = lax.fori_loop(0, 16, row, jnp.zeros((16,), jnp.float32))
            svall[k, pl.ds(gi * 16, 16)] = vec
            return 0
        lax.fori_loop(0, CH // 16, grp, 0)

    ga(0, 0)
    gb(0, 0)

    def body(k2, _):
        for b in (0, 1):
            k = 2 * k2 + b
            wait_ga(b)
            wait_gb(b)
            nk = k + 1

            @pl.when(nk < NCHUNK)
            def _():
                ga(nk, 1 - b)
                gb(nk, 1 - b)
            compute(k, b)
        return 0
    lax.fori_loop(0, NCHUNK // 2, body, 0)
    if NCHUNK % 2:
        wait_ga(0)
        wait_gb(0)
        compute(NCHUNK - 1, 0)
    pltpu.sync_copy(svall, out_hbm.at[wid])


_dec_kernel = functools.partial(
    pl.kernel,
    out_type=jax.ShapeDtypeStruct((NW, NCHUNK, CH), jnp.float32),
    mesh=_mesh,
    scratch_types=[
        pltpu.VMEM((NCHUNK, CH), jnp.int32),
        pltpu.VMEM((NCHUNK, CH), jnp.int32),
        pltpu.VMEM((2, CH, D), jnp.float32),
        pltpu.VMEM((2, CH, D), jnp.float32),
        pltpu.VMEM((NCHUNK, CH), jnp.float32),
        [pltpu.SemaphoreType.DMA] * 2,
        [pltpu.SemaphoreType.DMA] * 2,
    ],
)(_dec_body)


_BT = 1000


def _dinv_from(degT_blk):
    deg = jnp.sum(degT_blk, axis=1, keepdims=True) + 1.0
    return lax.rsqrt(deg)


def _tc1_body(degT_ref, x_ref, w1t_ref, b1_ref, g1_ref):
    dinv = _dinv_from(degT_ref[...])
    h = jnp.dot(x_ref[...], w1t_ref[...],
                preferred_element_type=jnp.float32) + b1_ref[...]
    g1_ref[...] = h * dinv


def _tc2_body(degT_ref, p0_ref, p1_ref, g1_ref, w2t_ref, b2_ref, g2_ref):
    dinv = _dinv_from(degT_ref[...])
    x1 = jnp.maximum((p0_ref[...] + p1_ref[...] + g1_ref[...]) * dinv, 0.0)
    h = jnp.dot(x1, w2t_ref[...],
                preferred_element_type=jnp.float32) + b2_ref[...]
    g2_ref[...] = h * dinv


def _tc3_body(degT_ref, p0_ref, p1_ref, g2_ref, z_ref):
    dinv = _dinv_from(degT_ref[...])
    z_ref[...] = jnp.maximum((p0_ref[...] + p1_ref[...] + g2_ref[...]) * dinv,
                             0.0)


def _row_spec(cols):
    return pl.BlockSpec((_BT, cols), lambda i: (i, 0))


def _full_spec(shape):
    return pl.BlockSpec(shape, lambda i: (0,) * len(shape))


def _tc1(degT, x, w1t, b1r):
    return pl.pallas_call(
        _tc1_body,
        grid=(N // _BT,),
        in_specs=[_row_spec(NC), _row_spec(D), _full_spec((D, D)),
                  _full_spec((1, D))],
        out_specs=_row_spec(D),
        out_shape=jax.ShapeDtypeStruct((N, D), jnp.float32),
    )(degT, x, w1t, b1r)


def _tc2(degT, p0, p1, g1, w2t, b2r):
    return pl.pallas_call(
        _tc2_body,
        grid=(N // _BT,),
        in_specs=[_row_spec(NC), _row_spec(D), _row_spec(D), _row_spec(D),
                  _full_spec((D, D)), _full_spec((1, D))],
        out_specs=_row_spec(D),
        out_shape=jax.ShapeDtypeStruct((N, D), jnp.float32),
    )(degT, p0, p1, g1, w2t, b2r)


def _tc3(degT, p0, p1, g2):
    return pl.pallas_call(
        _tc3_body,
        grid=(N // _BT,),
        in_specs=[_row_spec(NC), _row_spec(D), _row_spec(D), _row_spec(D)],
        out_specs=_row_spec(D),
        out_shape=jax.ShapeDtypeStruct((N, D), jnp.float32),
    )(degT, p0, p1, g2)


def kernel(x, edge_index, W1, b1, W2, b2):
    src = edge_index[0].astype(jnp.int32)
    dst = edge_index[1].astype(jnp.int32)
    src3 = src.reshape(NW, NCHUNK, CH)
    dst3 = dst.reshape(NW, NCHUNK, CH)
    src4 = src.reshape(NW, SNBLK, SIBL, SCH)
    dst4 = dst.reshape(NW, SNBLK, SIBL, SCH)

    degp = _deg_kernel(dst3)
    degT = degp.reshape(NC, NP)[:, :N].T

    g1 = _tc1(degT, x, W1.T, b1.reshape(1, D))
    parts1 = _scat_kernel(g1, src4, dst4)
    g2 = _tc2(degT, parts1[0], parts1[1], g1, W2.T, b2.reshape(1, D))
    parts2 = _scat_kernel(g2, src4, dst4)
    z = _tc3(degT, parts2[0], parts2[1], g2)
    scores = _dec_kernel(z, src3, dst3)
    return scores.reshape(E)


# speedup vs baseline: 1.0022x; 1.0022x over previous
import functools

import jax
import jax.numpy as jnp
from jax import lax
from jax.experimental import pallas as pl
from jax.experimental.pallas import tpu as pltpu
from jax.experimental.pallas import tpu_sc as plsc

N = 10000
E = 320000
D = 128

NC = 2
NS = 16
NW = NC * NS
EPW = E // NW
CH = 80
NCHUNK = EPW // CH
SCH = 125
SNCH = EPW // SCH
SIBL = 20
SNBLK = SNCH // SIBL
NP = 10240
DCH = 2560
ZR = CH
_NZB = -(-(N // ZR) // NS)

_mesh = plsc.VectorSubcoreMesh(
    core_axis_name="c", subcore_axis_name="s", num_cores=NC, num_subcores=NS)


def _deg_body(dst_hbm, degp_hbm, dstv2, onesv, zv, acc, sd):
    c = lax.axis_index("c")
    s = lax.axis_index("s")
    wid = c * NS + s
    zz = jnp.zeros((16,), jnp.float32)
    oo = jnp.ones((16,), jnp.float32)

    def fill(i, _):
        zv[pl.ds(i * 16, 16)] = zz
        return 0
    lax.fori_loop(0, DCH // 16, fill, 0)

    def fillo(i, _):
        onesv[pl.ds(i * 16, 16)] = oo
        return 0
    lax.fori_loop(0, CH // 16, fillo, 0)

    pltpu.sync_copy(dst_hbm.at[wid], dstv2)

    @pl.when(s < NP // DCH)
    def _():
        pltpu.sync_copy(zv, acc.at[pl.ds(s * DCH, DCH)])
    plsc.subcore_barrier()

    GRP = 25
    for g in range(NCHUNK // GRP):
        cps = [pltpu.async_copy(onesv, acc.at[dstv2.at[g * GRP + j]], sd,
                                add=True)
               for j in range(GRP)]
        for cp in cps:
            cp.wait()
    plsc.subcore_barrier()

    @pl.when(s < NP // DCH)
    def _():
        pltpu.sync_copy(acc.at[pl.ds(s * DCH, DCH)], zv)
        pltpu.sync_copy(zv, degp_hbm.at[pl.ds(c * NP + s * DCH, DCH)])


_deg_kernel = functools.partial(
    pl.kernel,
    out_type=jax.ShapeDtypeStruct((NC * NP,), jnp.float32),
    mesh=_mesh,
    scratch_types=[
        pltpu.VMEM((NCHUNK, CH), jnp.int32),
        pltpu.VMEM((CH,), jnp.float32),
        pltpu.VMEM((DCH,), jnp.float32),
        pltpu.VMEM_SHARED((NP,), jnp.float32),
        pltpu.SemaphoreType.DMA,
    ],
)(_deg_body)


_NB = 2


def _scat_body(g_hbm, src_hbm, dst_hbm, parts_hbm, srcv2, dstv2, rows,
               acc, sg, ss):
    c = lax.axis_index("c")
    s = lax.axis_index("s")
    wid = c * NS + s
    zz = jnp.zeros((16,), jnp.float32)

    def zero(i, _):
        rows[0, i // 8, pl.ds((i % 8) * 16, 16)] = zz
        return 0
    lax.fori_loop(0, ZR * 8, zero, 0)
    for j in range(_NZB):
        blk = s + NS * j

        @pl.when(blk < N // ZR)
        def _():
            pltpu.sync_copy(rows.at[0, pl.ds(0, ZR)],
                            acc.at[pl.ds(blk * ZR, ZR)])
    plsc.subcore_barrier()

    def gather(lk, b):
        return pltpu.async_copy(g_hbm.at[srcv2.at[lk]], rows.at[b], sg[b])

    def scat(lk, b):
        return pltpu.async_copy(rows.at[b], acc.at[dstv2.at[lk]], ss[b],
                                add=True)

    def wait_g(b):
        pltpu.make_async_copy(g_hbm.at[srcv2.at[0]], rows.at[b], sg[b]).wait()

    def wait_s(b):
        pltpu.make_async_copy(rows.at[b], acc.at[dstv2.at[0]], ss[b]).wait()

    for blk in range(SNBLK):
        pltpu.sync_copy(src_hbm.at[wid, blk], srcv2)
        pltpu.sync_copy(dst_hbm.at[wid, blk], dstv2)
        for b in range(_NB):
            gather(b, b)

        def body(k2, _):
            lk = k2 * _NB
            for b in range(_NB):
                wait_g(b)
                scat(lk + b, b)
                wait_s(b)
                nk = lk + _NB + b

                @pl.when(nk < SIBL)
                def _():
                    gather(nk, b)
            return 0
        lax.fori_loop(0, SIBL // _NB, body, 0)
        for t in range(SIBL - (SIBL // _NB) * _NB):
            wait_g(t)
            scat((SIBL // _NB) * _NB + t, t)
            wait_s(t)
    plsc.subcore_barrier()
    for j in range(_NZB):
        blk = s + NS * j

        @pl.when(blk < N // ZR)
        def _():
            pltpu.sync_copy(acc.at[pl.ds(blk * ZR, ZR)],
                            parts_hbm.at[c, pl.ds(blk * ZR, ZR)])


_scat_kernel = functools.partial(
    pl.kernel,
    out_type=jax.ShapeDtypeStruct((NC, N, D), jnp.float32),
    mesh=_mesh,
    scratch_types=[
        pltpu.VMEM((SIBL, SCH), jnp.int32),
        pltpu.VMEM((SIBL, SCH), jnp.int32),
        pltpu.VMEM((_NB, SCH, D), jnp.float32),
        pltpu.VMEM_SHARED((N, D), jnp.float32),
        [pltpu.SemaphoreType.DMA] * _NB,
        [pltpu.SemaphoreType.DMA] * _NB,
    ],
)(_scat_body)


def _dec_body(z_hbm, src_hbm, dst_hbm, out_hbm, srcv2, dstv2, za, zb, svall,
              sa, sb):
    c = lax.axis_index("c")
    s = lax.axis_index("s")
    wid = c * NS + s
    lane = lax.iota(jnp.int32, 16)

    pltpu.sync_copy(src_hbm.at[wid], srcv2)
    pltpu.sync_copy(dst_hbm.at[wid], dstv2)

    def ga(k, b):
        return pltpu.async_copy(z_hbm.at[srcv2.at[k]], za.at[b], sa[b])

    def gb(k, b):
        return pltpu.async_copy(z_hbm.at[dstv2.at[k]], zb.at[b], sb[b])

    def wait_ga(b):
        pltpu.make_async_copy(z_hbm.at[srcv2.at[0]], za.at[b], sa[b]).wait()

    def wait_gb(b):
        pltpu.make_async_copy(z_hbm.at[dstv2.at[0]], zb.at[b], sb[b]).wait()

    rots = [(lane + t) & 15 for t in (8, 4, 2, 1)]

    def compute(k, b):
        def grp(gi, _):
            def row(r16, vec):
                r = gi * 16 + r16
                p = za[b, r, pl.ds(0, 16)] * zb[b, r, pl.ds(0, 16)]
                for j in range(1, 8):
                    p = p + (za[b, r, pl.ds(j * 16, 16)]
                             * zb[b, r, pl.ds(j * 16, 16)])
                for rot in rots:
                    p = p + p.at[rot].get(mode="promise_in_bounds")
                return jnp.where(lane == r16, p, vec)
            vec = lax.fori_loop(0, 16, row, jnp.zeros((16,), jnp.float32))
            svall[k, pl.ds(gi * 16, 16)] = vec
            return 0
        lax.fori_loop(0, CH // 16, grp, 0)

    ga(0, 0)
    gb(0, 0)

    def body(k2, _):
        for b in (0, 1):
            k = 2 * k2 + b
            wait_ga(b)
            wait_gb(b)
            nk = k + 1

            @pl.when(nk < NCHUNK)
            def _():
                ga(nk, 1 - b)
                gb(nk, 1 - b)
            compute(k, b)
        return 0
    lax.fori_loop(0, NCHUNK // 2, body, 0)
    if NCHUNK % 2:
        wait_ga(0)
        wait_gb(0)
        compute(NCHUNK - 1, 0)
    pltpu.sync_copy(svall, out_hbm.at[wid])


_dec_kernel = functools.partial(
    pl.kernel,
    out_type=jax.ShapeDtypeStruct((NW, NCHUNK, CH), jnp.float32),
    mesh=_mesh,
    scratch_types=[
        pltpu.VMEM((NCHUNK, CH), jnp.int32),
        pltpu.VMEM((NCHUNK, CH), jnp.int32),
        pltpu.VMEM((2, CH, D), jnp.float32),
        pltpu.VMEM((2, CH, D), jnp.float32),
        pltpu.VMEM((NCHUNK, CH), jnp.float32),
        [pltpu.SemaphoreType.DMA] * 2,
        [pltpu.SemaphoreType.DMA] * 2,
    ],
)(_dec_body)


_BT = 1000


def _dinv_from(degT_blk):
    deg = jnp.sum(degT_blk, axis=1, keepdims=True) + 1.0
    return lax.rsqrt(deg)


def _tc1_body(degT_ref, x_ref, w1t_ref, b1_ref, g1_ref):
    dinv = _dinv_from(degT_ref[...])
    h = jnp.dot(x_ref[...], w1t_ref[...],
                preferred_element_type=jnp.float32) + b1_ref[...]
    g1_ref[...] = h * dinv


def _tc2_body(degT_ref, p0_ref, p1_ref, g1_ref, w2t_ref, b2_ref, g2_ref):
    dinv = _dinv_from(degT_ref[...])
    x1 = jnp.maximum((p0_ref[...] + p1_ref[...] + g1_ref[...]) * dinv, 0.0)
    h = jnp.dot(x1, w2t_ref[...],
                preferred_element_type=jnp.float32) + b2_ref[...]
    g2_ref[...] = h * dinv


def _tc3_body(degT_ref, p0_ref, p1_ref, g2_ref, z_ref):
    dinv = _dinv_from(degT_ref[...])
    z_ref[...] = jnp.maximum((p0_ref[...] + p1_ref[...] + g2_ref[...]) * dinv,
                             0.0)


def _row_spec(cols):
    return pl.BlockSpec((_BT, cols), lambda i: (i, 0))


def _full_spec(shape):
    return pl.BlockSpec(shape, lambda i: (0,) * len(shape))


def _tc1(degT, x, w1t, b1r):
    return pl.pallas_call(
        _tc1_body,
        grid=(N // _BT,),
        in_specs=[_row_spec(NC), _row_spec(D), _full_spec((D, D)),
                  _full_spec((1, D))],
        out_specs=_row_spec(D),
        out_shape=jax.ShapeDtypeStruct((N, D), jnp.float32),
    )(degT, x, w1t, b1r)


def _tc2(degT, p0, p1, g1, w2t, b2r):
    return pl.pallas_call(
        _tc2_body,
        grid=(N // _BT,),
        in_specs=[_row_spec(NC), _row_spec(D), _row_spec(D), _row_spec(D),
                  _full_spec((D, D)), _full_spec((1, D))],
        out_specs=_row_spec(D),
        out_shape=jax.ShapeDtypeStruct((N, D), jnp.float32),
    )(degT, p0, p1, g1, w2t, b2r)


def _tc3(degT, p0, p1, g2):
    return pl.pallas_call(
        _tc3_body,
        grid=(N // _BT,),
        in_specs=[_row_spec(NC), _row_spec(D), _row_spec(D), _row_spec(D)],
        out_specs=_row_spec(D),
        out_shape=jax.ShapeDtypeStruct((N, D), jnp.float32),
    )(degT, p0, p1, g2)


def kernel(x, edge_index, W1, b1, W2, b2):
    src = edge_index[0].astype(jnp.int32)
    dst = edge_index[1].astype(jnp.int32)
    src3 = src.reshape(NW, NCHUNK, CH)
    dst3 = dst.reshape(NW, NCHUNK, CH)
    src4 = src.reshape(NW, SNBLK, SIBL, SCH)
    dst4 = dst.reshape(NW, SNBLK, SIBL, SCH)

    degp = _deg_kernel(dst3)
    degT = degp.reshape(NC, NP)[:, :N].T

    g1 = _tc1(degT, x, W1.T, b1.reshape(1, D))
    parts1 = _scat_kernel(g1, src4, dst4)
    g2 = _tc2(degT, parts1[0], parts1[1], g1, W2.T, b2.reshape(1, D))
    parts2 = _scat_kernel(g2, src4, dst4)
    z = _tc3(degT, parts2[0], parts2[1], g2)
    scores = _dec_kernel(z, src3, dst3)
    return scores.reshape(E)
